# Initial kernel scaffold; baseline (speedup 1.0000x reference)
import jax
import jax.numpy as jnp
from jax.experimental import pallas as pl


def kernel(feats, edge_index, etype, V1, comp1, bias1, loopw1, V2, comp2, bias2, loopw2):
    raise NotImplementedError("write your pallas kernel here")


# speedup vs baseline: 68.1224x; 68.1224x over previous
import functools

import jax
import jax.numpy as jnp
from jax import lax
from jax.experimental import pallas as pl
from jax.experimental.pallas import tpu as pltpu
from jax.experimental.pallas import tpu_sc as plsc

N_NODES = 10000
N_EDGES = 160000
D = 256
R = 8
NB = 4
NRP = R + 1
NH = 2
NC, NS = 2, 16
HALF = N_NODES // NC
HP = 5120
ACC_T = NH * HP // NS
DP = 10112
DEG_T = DP // NS
DEG_E = 5120
CHUNK = 128
NCHUNK = 80
TILE_E = NCHUNK * CHUNK
E_PAD = NS * TILE_E
BN = 400
NBLK = N_NODES // BN
_PAD_ROWS = E_PAD // 128
_VALID_ROWS = N_EDGES // 128


def _c(*vals):
    return tuple(jnp.int32(v) for v in vals)


def _m00(i):
    return _c(0, 0)


def _m000(i):
    return _c(0, 0, 0)


def _mi0(i):
    return (i, jnp.int32(0))


def _mdeg0(i):
    return (jnp.int32(0), i, jnp.int32(0))


def _mdeg1(i):
    return (jnp.int32(1), i, jnp.int32(0))


def _mhl0(i):
    return (i, jnp.int32(R))


def _mhl1(i):
    return (i, jnp.int32(NRP + R))


def _sc_agg_body(table, ga4, gb4, sa4, sb4, zacc,
                 agg_out,
                 acc_sh, gi0, gi1, si0, si1, ra0, ra1,
                 gs0, gs1, ss0, ss1, ia0, ia1, iw0, iw1):
    cid = lax.axis_index("c")
    sid = lax.axis_index("s")
    pltpu.sync_copy(zacc.at[pl.ds(sid * ACC_T, ACC_T)],
                    acc_sh.at[pl.ds(sid * ACC_T, ACC_T)])
    plsc.subcore_barrier()

    gi = (gi0, gi1)
    si = (si0, si1)
    ra = (ra0, ra1)
    gs = (gs0, gs1)
    ss = (ss0, ss1)
    ia = (ia0, ia1)
    iw = (iw0, iw1)

    def run_phase(g4, s4):
        pltpu.sync_copy(g4.at[sid, 0], gi0)
        pltpu.async_copy(s4.at[cid, sid, 0], si0, iw0)
        pltpu.async_copy(table.at[gi0], ra0, gs0)
        pltpu.async_copy(g4.at[sid, 1], gi1, ia1)

        def step(j, carry):
            for b in range(2):
                i = j * 2 + b
                pltpu.make_async_copy(table.at[gi[b]], ra[b], gs[b]).wait()

                @pl.when(i + 2 < NCHUNK)
                def _ga_prefetch():
                    pltpu.async_copy(g4.at[sid, i + 2], gi[b], ia[b])

                @pl.when(i >= 1)
                def _wait_prev_scatter():
                    pltpu.make_async_copy(
                        ra[1 - b], acc_sh.at[si[1 - b]], ss[1 - b]).wait()

                @pl.when(i + 1 < NCHUNK)
                def _sa_prefetch():
                    pltpu.async_copy(s4.at[cid, sid, i + 1], si[1 - b],
                                     iw[1 - b])

                @pl.when(i + 1 < NCHUNK)
                def _gather_next():
                    pltpu.make_async_copy(g4.at[sid, i + 1], gi[1 - b],
                                          ia[1 - b]).wait()
                    pltpu.async_copy(table.at[gi[1 - b]], ra[1 - b],
                                     gs[1 - b])

                pltpu.make_async_copy(s4.at[cid, sid, i], si[b], iw[b]).wait()
                pltpu.async_copy(ra[b], acc_sh.at[si[b]], ss[b], add=True)
            return carry

        lax.fori_loop(jnp.int32(0), jnp.int32(NCHUNK // 2), step,
                      jnp.int32(0))
        pltpu.make_async_copy(ra1, acc_sh.at[si1], ss1).wait()

    run_phase(ga4, sa4)
    run_phase(gb4, sb4)
    plsc.subcore_barrier()
    pltpu.sync_copy(acc_sh.at[pl.ds(sid * ACC_T, ACC_T)],
                    agg_out.at[pl.ds(cid * NH * HP + sid * ACC_T, ACC_T)])


_sc_agg = pl.kernel(
    _sc_agg_body,
    out_type=jax.ShapeDtypeStruct((NC * NH * HP, 128), jnp.float32),
    mesh=plsc.VectorSubcoreMesh(core_axis_name="c", subcore_axis_name="s"),
    scratch_types=[
        pltpu.VMEM_SHARED((NH * HP, 128), jnp.float32),
        pltpu.VMEM((CHUNK,), jnp.int32),
        pltpu.VMEM((CHUNK,), jnp.int32),
        pltpu.VMEM((CHUNK,), jnp.int32),
        pltpu.VMEM((CHUNK,), jnp.int32),
        pltpu.VMEM((CHUNK, 128), jnp.float32),
        pltpu.VMEM((CHUNK, 128), jnp.float32),
        pltpu.SemaphoreType.DMA,
        pltpu.SemaphoreType.DMA,
        pltpu.SemaphoreType.DMA,
        pltpu.SemaphoreType.DMA,
        pltpu.SemaphoreType.DMA,
        pltpu.SemaphoreType.DMA,
        pltpu.SemaphoreType.DMA,
        pltpu.SemaphoreType.DMA,
    ],
)


def _sc_deg_body(dstp, zdeg, onesc,
                 deg_out,
                 deg_sh, dst_v, ones_v):
    cid = lax.axis_index("c")
    sid = lax.axis_index("s")
    pltpu.sync_copy(zdeg.at[pl.ds(sid * DEG_T, DEG_T)],
                    deg_sh.at[pl.ds(sid * DEG_T, DEG_T)])
    pltpu.sync_copy(onesc, ones_v)
    plsc.subcore_barrier()
    w = cid * NS + sid

    def step(i, carry):
        base = w * DEG_E + i * CHUNK
        pltpu.sync_copy(dstp.at[pl.ds(base, CHUNK)], dst_v)
        pltpu.sync_copy(ones_v, deg_sh.at[dst_v], add=True)
        return carry

    lax.fori_loop(jnp.int32(0), jnp.int32(DEG_E // CHUNK), step,
                  jnp.int32(0))
    plsc.subcore_barrier()
    pltpu.sync_copy(deg_sh.at[pl.ds(sid * DEG_T, DEG_T)],
                    deg_out.at[cid, pl.ds(sid * DEG_T, DEG_T)])


_sc_deg = pl.kernel(
    _sc_deg_body,
    out_type=jax.ShapeDtypeStruct((NC, DP, 128), jnp.float32),
    mesh=plsc.VectorSubcoreMesh(core_axis_name="c", subcore_axis_name="s"),
    scratch_types=[
        pltpu.VMEM_SHARED((DP, 128), jnp.float32),
        pltpu.VMEM((CHUNK,), jnp.int32),
        pltpu.VMEM((CHUNK, 128), jnp.float32),
    ],
)


def _prep_body(src_ref, et_ref, dst_ref,
               ga_ref, gb_ref, sa_ref, sb_ref, dstp_ref,
               zacc_ref, zdeg_ref, ones_ref):
    rows_i = lax.broadcasted_iota(jnp.int32, (_PAD_ROWS, 128), 0)
    lane_i = lax.broadcasted_iota(jnp.int32, (_PAD_ROWS, 128), 1)
    pad = rows_i >= _VALID_ROWS
    src = src_ref[...]
    et = et_ref[...]
    ga = jnp.where(pad, lane_i * (NH * NRP), src * (NH * NRP) + et)
    ga_ref[...] = ga
    gb_ref[...] = ga + NRP
    d = dst_ref[...]
    dum = HALF + lane_i % (HP - HALF)
    s0 = jnp.where(pad | (d >= HALF), dum, d)
    s1 = jnp.where(pad | (d < HALF), dum, d - HALF)
    sa_ref[0] = s0
    sa_ref[1] = s1
    sb_ref[0] = s0 + HP
    sb_ref[1] = s1 + HP
    dstp_ref[...] = jnp.where(pad, N_NODES + lane_i % (DP - N_NODES), d)
    zacc_ref[...] = jnp.zeros((NH * HP, 128), jnp.float32)
    zdeg_ref[...] = jnp.zeros((DP, 128), jnp.float32)
    ones_ref[...] = jnp.ones((CHUNK, 128), jnp.float32)


_prep = pl.pallas_call(
    _prep_body,
    grid=(1,),
    in_specs=[pl.BlockSpec((_PAD_ROWS, 128), _m00),
              pl.BlockSpec((_PAD_ROWS, 128), _m00),
              pl.BlockSpec((_PAD_ROWS, 128), _m00)],
    out_specs=[pl.BlockSpec((_PAD_ROWS, 128), _m00),
               pl.BlockSpec((_PAD_ROWS, 128), _m00),
               pl.BlockSpec((2, _PAD_ROWS, 128), _m000),
               pl.BlockSpec((2, _PAD_ROWS, 128), _m000),
               pl.BlockSpec((_PAD_ROWS, 128), _m00),
               pl.BlockSpec((NH * HP, 128), _m00),
               pl.BlockSpec((DP, 128), _m00),
               pl.BlockSpec((CHUNK, 128), _m00)],
    out_shape=[jax.ShapeDtypeStruct((_PAD_ROWS, 128), jnp.int32),
               jax.ShapeDtypeStruct((_PAD_ROWS, 128), jnp.int32),
               jax.ShapeDtypeStruct((2, _PAD_ROWS, 128), jnp.int32),
               jax.ShapeDtypeStruct((2, _PAD_ROWS, 128), jnp.int32),
               jax.ShapeDtypeStruct((_PAD_ROWS, 128), jnp.int32),
               jax.ShapeDtypeStruct((NH * HP, 128), jnp.float32),
               jax.ShapeDtypeStruct((DP, 128), jnp.float32),
               jax.ShapeDtypeStruct((CHUNK, 128), jnp.float32)],
)


def _wcat_body(comp_ref, v_ref, loopw_ref, out_ref):
    for r in range(R):
        acc = comp_ref[r, 0] * v_ref[0]
        for b in range(1, NB):
            acc = acc + comp_ref[r, b] * v_ref[b]
        for h in range(NH):
            out_ref[:, (h * NRP + r) * 128:(h * NRP + r + 1) * 128] = (
                acc[:, h * 128:(h + 1) * 128])
    lw = loopw_ref[...]
    for h in range(NH):
        out_ref[:, (h * NRP + R) * 128:(h * NRP + R + 1) * 128] = (
            lw[:, h * 128:(h + 1) * 128])


_wcat = pl.pallas_call(
    _wcat_body,
    grid=(1,),
    in_specs=[
        pl.BlockSpec(memory_space=pltpu.SMEM),
        pl.BlockSpec((NB, D, D), _m000),
        pl.BlockSpec((D, D), _m00),
    ],
    out_specs=pl.BlockSpec((D, NH * NRP * 128), _m00),
    out_shape=jax.ShapeDtypeStruct((D, NH * NRP * 128), jnp.float32),
)


def _mm_body(h_ref, w_ref, o_ref):
    o_ref[...] = jnp.dot(h_ref[...], w_ref[...],
                         preferred_element_type=jnp.float32)


_mm = pl.pallas_call(
    _mm_body,
    grid=(NBLK,),
    in_specs=[pl.BlockSpec((BN, D), _mi0),
              pl.BlockSpec((D, NH * NRP * 128), _m00)],
    out_specs=pl.BlockSpec((BN, NH * NRP * 128), _mi0),
    out_shape=jax.ShapeDtypeStruct((N_NODES, NH * NRP * 128), jnp.float32),
)


def _fin_body(a0_ref, a1_ref, d0_ref, d1_ref, bias_ref, hl0_ref, hl1_ref,
              o_ref, *, act):
    d = d0_ref[0][:, 0:1] + d1_ref[0][:, 0:1]
    inv = 1.0 / jnp.maximum(d, 1.0)
    o0 = a0_ref[...] * inv + bias_ref[:, 0:128] + hl0_ref[...]
    o1 = a1_ref[...] * inv + bias_ref[:, 128:256] + hl1_ref[...]
    if act:
        o0 = jnp.maximum(o0, 0.0)
        o1 = jnp.maximum(o1, 0.0)
    o_ref[:, 0:128] = o0
    o_ref[:, 128:256] = o1


def _make_fin(act):
    return pl.pallas_call(
        functools.partial(_fin_body, act=act),
        grid=(NBLK,),
        in_specs=[pl.BlockSpec((BN, 128), _mi0),
                  pl.BlockSpec((BN, 128), _mi0),
                  pl.BlockSpec((1, BN, 128), _mdeg0),
                  pl.BlockSpec((1, BN, 128), _mdeg1),
                  pl.BlockSpec((1, D), _m00),
                  pl.BlockSpec((BN, 128), _mhl0),
                  pl.BlockSpec((BN, 128), _mhl1)],
        out_specs=pl.BlockSpec((BN, D), _mi0),
        out_shape=jax.ShapeDtypeStruct((N_NODES, D), jnp.float32),
    )


_fin_relu = _make_fin(True)
_fin_lin = _make_fin(False)


def kernel(feats, edge_index, etype, V1, comp1, bias1, loopw1,
           V2, comp2, bias2, loopw2):
    with jax.enable_x64(False):
        h2 = _kernel_f32(feats, edge_index, etype, V1, comp1, bias1, loopw1,
                         V2, comp2, bias2, loopw2)
    return h2.astype(jnp.float64)


def _kernel_f32(feats, edge_index, etype, V1, comp1, bias1, loopw1,
                V2, comp2, bias2, loopw2):
    f32 = jnp.float32
    feats = feats.astype(f32)
    V1, comp1, bias1, loopw1 = (x.astype(f32) for x in (V1, comp1, bias1, loopw1))
    V2, comp2, bias2, loopw2 = (x.astype(f32) for x in (V2, comp2, bias2, loopw2))
    src = edge_index[0].astype(jnp.int32)
    dst = edge_index[1].astype(jnp.int32)
    et = etype.astype(jnp.int32)

    src2d = jnp.pad(src, (0, E_PAD - N_EDGES)).reshape(_PAD_ROWS, 128)
    et2d = jnp.pad(et, (0, E_PAD - N_EDGES)).reshape(_PAD_ROWS, 128)
    dst2d = jnp.pad(dst, (0, E_PAD - N_EDGES)).reshape(_PAD_ROWS, 128)
    (ga2d, gb2d, sa3d, sb3d, dstp2d,
     zacc, zdeg, onesc) = _prep(src2d, et2d, dst2d)
    ga = ga2d.reshape(NS, NCHUNK, CHUNK)
    gb = gb2d.reshape(NS, NCHUNK, CHUNK)
    sa = sa3d.reshape(NC, NS, NCHUNK, CHUNK)
    sb = sb3d.reshape(NC, NS, NCHUNK, CHUNK)
    dstp = dstp2d.reshape(NC * NS * DEG_E)

    wcat1 = _wcat(comp1, V1, loopw1)
    wcat2 = _wcat(comp2, V2, loopw2)
    degout = _sc_deg(dstp, zdeg, onesc)

    def layer(h, wcat, bias, fin):
        p = _mm(h, wcat)
        aggout = _sc_agg(p.reshape(N_NODES * NH * NRP, 128),
                         ga, gb, sa, sb, zacc)
        agg0 = jnp.concatenate([aggout[0:HALF],
                                aggout[NH * HP:NH * HP + HALF]], axis=0)
        agg1 = jnp.concatenate([aggout[HP:HP + HALF],
                                aggout[NH * HP + HP:NH * HP + HP + HALF]],
                               axis=0)
        return fin(agg0, agg1, degout, degout, bias.reshape(1, D), p, p)

    h1 = layer(feats, wcat1, bias1, _fin_relu)
    return layer(h1, wcat2, bias2, _fin_lin)
